# Initial kernel scaffold; baseline (speedup 1.0000x reference)
import jax
import jax.numpy as jnp
from jax.experimental import pallas as pl


def kernel(input, indices):
    raise NotImplementedError("write your pallas kernel here")


# speedup vs baseline: 3.5135x; 3.5135x over previous
import functools

import jax
import jax.numpy as jnp
from jax import lax
from jax.experimental import pallas as pl
from jax.experimental.pallas import tpu as pltpu
from jax.experimental.pallas import tpu_sc as plsc

KH, KW = 2, 2
DH, DW = 2, 2


def _unpool_sc(vals2d, idx2d, n_planes, n_in, n_out):
    info = plsc.get_sparse_core_info()
    NC, NS, L = info.num_cores, info.num_subcores, info.num_lanes
    NW = NC * NS
    planes_per_w = n_planes // NW
    n_in_vecs = n_in // L
    n_out_vecs = n_out // L

    mesh = plsc.VectorSubcoreMesh(core_axis_name="c", subcore_axis_name="s")

    @functools.partial(
        pl.kernel,
        mesh=mesh,
        compiler_params=pltpu.CompilerParams(needs_layout_passes=False),
        out_type=jax.ShapeDtypeStruct((n_planes, n_out), jnp.float32),
        scratch_types=[
            pltpu.VMEM((n_in,), jnp.int32),
            pltpu.VMEM((n_in,), jnp.float32),
            pltpu.VMEM((n_out,), jnp.float32),
        ],
    )
    def k(vals_hbm, idx_hbm, out_hbm, idx_v, val_v, out_v):
        wid = lax.axis_index("s") * NC + lax.axis_index("c")

        zeros = jnp.zeros((L,), jnp.float32)

        def per_plane(kk, _):
            p = wid * planes_per_w + kk
            pltpu.sync_copy(idx_hbm.at[p], idx_v)
            pltpu.sync_copy(vals_hbm.at[p], val_v)

            def zero_body(i, c):
                out_v[pl.ds(i * L, L)] = zeros
                return c

            lax.fori_loop(0, n_out_vecs, zero_body, None)

            def scat_body(i, c):
                ids = idx_v[pl.ds(i * L, L)]
                vs = val_v[pl.ds(i * L, L)]
                plsc.store_scatter(out_v, [ids], vs)
                return c

            lax.fori_loop(0, n_in_vecs, scat_body, None)

            pltpu.sync_copy(out_v, out_hbm.at[p])
            return _

        lax.fori_loop(0, planes_per_w, per_plane, None)

    return k(vals2d, idx2d)


def kernel(input, indices):
    B, C, H, W = input.shape
    Hout = (H - 1) * DH + KH
    Wout = (W - 1) * DW + KW
    n_planes = B * C
    n_in = H * W
    n_out = Hout * Wout
    vals2d = input.reshape(n_planes, n_in)
    idx2d = indices.reshape(n_planes, n_in).astype(jnp.int32)
    plane_off = jnp.arange(n_planes, dtype=jnp.int32)[:, None] * n_out
    gkeys = (idx2d + plane_off).reshape(-1)
    sk, sv = jax.lax.sort((gkeys, vals2d.reshape(-1)), num_keys=1,
                          is_stable=False)
    lk2d = sk.reshape(n_planes, n_in) - plane_off
    sv2d = sv.reshape(n_planes, n_in)
    out = _unpool_sc(sv2d, lk2d, n_planes, n_in, n_out)
    return out.reshape(B, C, Hout, Wout)


# speedup vs baseline: 3.5218x; 1.0023x over previous
import functools

import jax
import jax.numpy as jnp
from jax import lax
from jax.experimental import pallas as pl
from jax.experimental.pallas import tpu as pltpu
from jax.experimental.pallas import tpu_sc as plsc

KH, KW = 2, 2
DH, DW = 2, 2


def _unpool_sc(sv, sk, n_planes, n_in, n_out):
    info = plsc.get_sparse_core_info()
    NC, NS, L = info.num_cores, info.num_subcores, info.num_lanes
    NW = NC * NS
    planes_per_w = n_planes // NW
    n_in_vecs = n_in // L
    n_out_vecs = n_out // L

    mesh = plsc.VectorSubcoreMesh(core_axis_name="c", subcore_axis_name="s")

    @functools.partial(
        pl.kernel,
        mesh=mesh,
        compiler_params=pltpu.CompilerParams(needs_layout_passes=False),
        out_type=jax.ShapeDtypeStruct((n_planes * n_out,), jnp.float32),
        scratch_types=[
            pltpu.VMEM((n_in,), jnp.int32),
            pltpu.VMEM((n_in,), jnp.float32),
            pltpu.VMEM((n_out,), jnp.float32),
        ],
    )
    def k(vals_hbm, idx_hbm, out_hbm, idx_v, val_v, out_v):
        wid = lax.axis_index("s") * NC + lax.axis_index("c")

        zeros = jnp.zeros((L,), jnp.float32)

        def per_plane(kk, _):
            p = wid * planes_per_w + kk
            base = p * n_out
            pltpu.sync_copy(idx_hbm.at[pl.ds(p * n_in, n_in)], idx_v)
            pltpu.sync_copy(vals_hbm.at[pl.ds(p * n_in, n_in)], val_v)

            def zero_body(i, c):
                out_v[pl.ds(i * L, L)] = zeros
                return c

            lax.fori_loop(0, n_out_vecs, zero_body, None)

            def scat_body(i, c):
                ids = idx_v[pl.ds(i * L, L)] - base
                vs = val_v[pl.ds(i * L, L)]
                plsc.store_scatter(out_v, [ids], vs)
                return c

            lax.fori_loop(0, n_in_vecs, scat_body, None)

            pltpu.sync_copy(out_v, out_hbm.at[pl.ds(base, n_out)])
            return _

        lax.fori_loop(0, planes_per_w, per_plane, None)

    return k(sv, sk)


def kernel(input, indices):
    B, C, H, W = input.shape
    Hout = (H - 1) * DH + KH
    Wout = (W - 1) * DW + KW
    n_planes = B * C
    n_in = H * W
    n_out = Hout * Wout
    vals2d = input.reshape(n_planes, n_in)
    idx2d = indices.reshape(n_planes, n_in).astype(jnp.int32)
    plane_off = jnp.arange(n_planes, dtype=jnp.int32)[:, None] * n_out
    gkeys = (idx2d + plane_off).reshape(-1)
    sk, sv = jax.lax.sort((gkeys, vals2d.reshape(-1)), num_keys=1,
                          is_stable=False)
    out = _unpool_sc(sv, sk, n_planes, n_in, n_out)
    return out.reshape(B, C, Hout, Wout)


# speedup vs baseline: 3.6312x; 1.0311x over previous
import functools

import jax
import jax.numpy as jnp
from jax import lax
from jax.experimental import pallas as pl
from jax.experimental.pallas import tpu as pltpu
from jax.experimental.pallas import tpu_sc as plsc

KH, KW = 2, 2
DH, DW = 2, 2


def _unpool_sc(sv, sk, n_planes, n_in, n_out):
    info = plsc.get_sparse_core_info()
    NC, NS, L = info.num_cores, info.num_subcores, info.num_lanes
    NW = NC * NS
    planes_per_w = n_planes // NW
    n_in_vecs = n_in // L
    n_out_vecs = n_out // L

    mesh = plsc.VectorSubcoreMesh(core_axis_name="c", subcore_axis_name="s")

    UZ = 8
    US = 4

    @functools.partial(
        pl.kernel,
        mesh=mesh,
        compiler_params=pltpu.CompilerParams(needs_layout_passes=False),
        out_type=jax.ShapeDtypeStruct((n_planes * n_out,), jnp.float32),
        scratch_types=[
            pltpu.VMEM((n_in,), jnp.int32),
            pltpu.VMEM((n_in,), jnp.float32),
            pltpu.VMEM((n_out,), jnp.float32),
            pltpu.VMEM((n_out,), jnp.float32),
            pltpu.SemaphoreType.DMA,
            pltpu.SemaphoreType.DMA,
            pltpu.SemaphoreType.DMA,
        ],
    )
    def k(vals_hbm, idx_hbm, out_hbm, idx_v, val_v, out_v0, out_v1,
          in_sem, osem0, osem1):
        wid = lax.axis_index("s") * NC + lax.axis_index("c")
        p_first = wid * planes_per_w

        zeros = jnp.zeros((L,), jnp.float32)

        def zero_buf(buf):
            def zb(i, c):
                for u in range(UZ):
                    buf[pl.ds(i * (L * UZ) + u * L, L)] = zeros
                return c

            lax.fori_loop(0, n_out_vecs // UZ, zb, None)

        def scatter_into(buf, p):
            base = p * n_out

            def sb(i, c):
                for u in range(US):
                    off = i * (L * US) + u * L
                    ids = idx_v[pl.ds(off, L)] - base
                    vs = val_v[pl.ds(off, L)]
                    plsc.store_scatter(buf, [ids], vs)
                return c

            lax.fori_loop(0, n_in_vecs // US, sb, None)

        def fetch_in(p):
            pltpu.async_copy(idx_hbm.at[pl.ds(p * n_in, n_in)], idx_v, in_sem)
            pltpu.async_copy(vals_hbm.at[pl.ds(p * n_in, n_in)], val_v, in_sem)

        def wait_in():
            pltpu.make_async_copy(idx_hbm.at[pl.ds(0, n_in)], idx_v,
                                  in_sem).wait()
            pltpu.make_async_copy(vals_hbm.at[pl.ds(0, n_in)], val_v,
                                  in_sem).wait()

        def drain_out(buf, osem):
            pltpu.make_async_copy(buf, out_hbm.at[pl.ds(0, n_out)],
                                  osem).wait()

        fetch_in(p_first)

        def body(j, _):
            p0 = p_first + 2 * j

            @pl.when(j > 0)
            def _wait0():
                drain_out(out_v0, osem0)

            zero_buf(out_v0)
            wait_in()
            scatter_into(out_v0, p0)
            pltpu.async_copy(out_v0, out_hbm.at[pl.ds(p0 * n_out, n_out)],
                             osem0)
            fetch_in(p0 + 1)

            @pl.when(j > 0)
            def _wait1():
                drain_out(out_v1, osem1)

            zero_buf(out_v1)
            wait_in()
            scatter_into(out_v1, p0 + 1)
            pltpu.async_copy(out_v1,
                             out_hbm.at[pl.ds((p0 + 1) * n_out, n_out)],
                             osem1)

            @pl.when(j < planes_per_w // 2 - 1)
            def _prefetch_next():
                fetch_in(p0 + 2)

            return _

        lax.fori_loop(0, planes_per_w // 2, body, None)
        drain_out(out_v0, osem0)
        drain_out(out_v1, osem1)

    return k(sv, sk)


def kernel(input, indices):
    B, C, H, W = input.shape
    Hout = (H - 1) * DH + KH
    Wout = (W - 1) * DW + KW
    n_planes = B * C
    n_in = H * W
    n_out = Hout * Wout
    vals2d = input.reshape(n_planes, n_in)
    idx2d = indices.reshape(n_planes, n_in).astype(jnp.int32)
    plane_off = jnp.arange(n_planes, dtype=jnp.int32)[:, None] * n_out
    gkeys = (idx2d + plane_off).reshape(-1)
    sk, sv = jax.lax.sort((gkeys, vals2d.reshape(-1)), num_keys=1,
                          is_stable=False)
    out = _unpool_sc(sv, sk, n_planes, n_in, n_out)
    return out.reshape(B, C, Hout, Wout)
